# baseline (speedup 1.0000x reference)
import jax
import jax.numpy as jnp
from jax import lax
from jax.experimental import pallas as pl
from jax.experimental.pallas import tpu as pltpu
from jax.experimental.pallas import tpu_sc as plsc

NUM_GENES = 1000000
EMBED_DIM = 64
N_SAMPLES = 16384
N_GENES_PER = 50
SB = N_SAMPLES // 128
N_UNITS = N_GENES_PER * SB
NW = 32
PER_W = N_UNITS // NW
MAIN_PAIRS = (PER_W - 4) // 2


def _gather_kernel(idxt_hbm, table_hbm, out_hbm, idx0, idx1, rows0, rows1,
                   ot0, ot1, is0, is1, gs0, gs1, os0, os1):
    idxb = [idx0, idx1]
    rows = [rows0, rows1]
    ot = [ot0, ot1]
    isem = [is0, is1]
    gsem = [gs0, gs1]
    osem = [os0, os1]
    wid = lax.axis_index("s") * 2 + lax.axis_index("c")
    u_base = wid * PER_W
    iota16 = lax.iota(jnp.int32, 16)

    def unit_jsb(u):
        ug = u_base + u
        return ug // SB, ug % SB

    def idx_copy(u, s):
        j, sb = unit_jsb(u)
        pltpu.async_copy(idxt_hbm.at[j, pl.ds(sb * 128, 128)], idxb[s],
                         isem[s])

    def idx_wait(u, s):
        j, sb = unit_jsb(u)
        pltpu.make_async_copy(idxt_hbm.at[j, pl.ds(sb * 128, 128)], idxb[s],
                              isem[s]).wait()

    def gather(s, b):
        pltpu.async_copy(table_hbm.at[idxb[s]], rows[b], gsem[b])

    def gather_wait(s, b):
        pltpu.make_async_copy(table_hbm.at[idxb[s]], rows[b], gsem[b]).wait()

    rk = [k0 * 16 + iota16 for k0 in range(8)]

    def transpose(b):
        def body_e0(e0, carry):
            for eo in range(8):
                e = e0 * 8 + eo
                col = jnp.full((16,), e, jnp.int32)
                for k0 in range(8):
                    v = plsc.load_gather(rows[b], [rk[k0], col])
                    ot[b][e, pl.ds(k0 * 16, 16)] = v
            return carry
        lax.fori_loop(0, EMBED_DIM // 8, body_e0, 0)

    def store(u, b):
        j, sb = unit_jsb(u)
        for eb in range(8):
            pltpu.async_copy(ot[b].at[pl.ds(eb * 8, 8)],
                             out_hbm.at[j, eb, sb], osem[b])

    def store_wait(u, b):
        j, sb = unit_jsb(u)
        for eb in range(8):
            pltpu.make_async_copy(ot[b].at[pl.ds(eb * 8, 8)],
                                  out_hbm.at[j, eb, sb], osem[b]).wait()

    idx_copy(0, 0)
    idx_copy(1, 1)
    idx_wait(0, 0)
    gather(0, 0)
    idx_wait(1, 1)
    gather(1, 1)
    gather_wait(0, 0)
    idx_copy(2, 0)
    transpose(0)
    store(0, 0)
    idx_wait(2, 0)
    gather(0, 0)
    gather_wait(1, 1)
    idx_copy(3, 1)
    transpose(1)
    store(1, 1)

    def body(i, carry):
        u = 2 * i + 2
        idx_wait(u + 1, 1)
        gather(1, 1)
        gather_wait(0, 0)
        idx_copy(u + 2, 0)
        store_wait(u - 2, 0)
        transpose(0)
        store(u, 0)
        idx_wait(u + 2, 0)
        gather(0, 0)
        gather_wait(1, 1)
        idx_copy(u + 3, 1)
        store_wait(u - 1, 1)
        transpose(1)
        store(u + 1, 1)
        return carry

    lax.fori_loop(0, MAIN_PAIRS, body, 0)

    u = PER_W - 2
    idx_wait(u + 1, 1)
    gather(1, 1)
    gather_wait(0, 0)
    store_wait(u - 2, 0)
    transpose(0)
    store(u, 0)
    gather_wait(1, 1)
    store_wait(u - 1, 1)
    transpose(1)
    store(u + 1, 1)
    store_wait(u, 0)
    store_wait(u + 1, 1)


@jax.jit
def _embed(gene_idx, table):
    idxt = gene_idx.T
    mesh = plsc.VectorSubcoreMesh(core_axis_name="c", subcore_axis_name="s")
    k = pl.kernel(
        _gather_kernel,
        mesh=mesh,
        out_type=jax.ShapeDtypeStruct((N_GENES_PER, 8, SB, 8, 128),
                                      jnp.float32),
        scratch_types=[
            pltpu.VMEM((128,), jnp.int32),
            pltpu.VMEM((128,), jnp.int32),
            pltpu.VMEM((128, EMBED_DIM), jnp.float32),
            pltpu.VMEM((128, EMBED_DIM), jnp.float32),
            pltpu.VMEM((EMBED_DIM, 128), jnp.float32),
            pltpu.VMEM((EMBED_DIM, 128), jnp.float32),
            pltpu.SemaphoreType.DMA,
            pltpu.SemaphoreType.DMA,
            pltpu.SemaphoreType.DMA,
            pltpu.SemaphoreType.DMA,
            pltpu.SemaphoreType.DMA,
            pltpu.SemaphoreType.DMA,
        ],
        compiler_params=pltpu.CompilerParams(
            use_tc_tiling_on_sc=False, needs_layout_passes=False),
    )(idxt, table)
    return k.transpose(2, 4, 0, 1, 3).reshape(N_SAMPLES, N_GENES_PER,
                                              EMBED_DIM)


def kernel(gene_idx, table):
    return _embed(gene_idx, table)


# speedup vs baseline: 1.4895x; 1.4895x over previous
import jax
import jax.numpy as jnp
from jax import lax
from jax.experimental import pallas as pl
from jax.experimental.pallas import tpu as pltpu
from jax.experimental.pallas import tpu_sc as plsc

NUM_GENES = 1000000
EMBED_DIM = 64
N_SAMPLES = 16384
N_GENES_PER = 50
B_TOTAL = N_SAMPLES * N_GENES_PER
NW = 32
SAMP_PER_W = N_SAMPLES // NW
PER_W = B_TOTAL // NW
SAMP_PER_CHUNK = 8
CHUNK = SAMP_PER_CHUNK * N_GENES_PER
N_CHUNKS = SAMP_PER_W // SAMP_PER_CHUNK
NBUF = 4
MAIN_ITERS = (N_CHUNKS - NBUF) // NBUF


def _gather_kernel(idx_hbm, table_hbm, out_hbm, idx_v, rows0, rows1, rows2,
                   rows3, gs0, gs1, gs2, gs3, os0, os1, os2, os3):
    rows = [rows0, rows1, rows2, rows3]
    gs = [gs0, gs1, gs2, gs3]
    osm = [os0, os1, os2, os3]
    wid = lax.axis_index("s") * 2 + lax.axis_index("c")
    samp_base = wid * SAMP_PER_W

    pltpu.sync_copy(idx_hbm.at[pl.ds(wid * PER_W, PER_W)], idx_v)

    def gather(c, b):
        src = table_hbm.at[idx_v.at[pl.ds(c * CHUNK, CHUNK)]]
        return pltpu.async_copy(src, rows[b], gs[b])

    def gather_wait(c, b):
        src = table_hbm.at[idx_v.at[pl.ds(c * CHUNK, CHUNK)]]
        pltpu.make_async_copy(src, rows[b], gs[b]).wait()

    def store(c, b):
        s0 = samp_base + c * SAMP_PER_CHUNK
        for k in range(SAMP_PER_CHUNK):
            pltpu.async_copy(
                rows[b].at[pl.ds(k * N_GENES_PER, N_GENES_PER)],
                out_hbm.at[s0 + k], osm[b])

    def store_wait(c, b):
        s0 = samp_base + c * SAMP_PER_CHUNK
        for k in range(SAMP_PER_CHUNK):
            pltpu.make_async_copy(
                rows[b].at[pl.ds(k * N_GENES_PER, N_GENES_PER)],
                out_hbm.at[s0 + k], osm[b]).wait()

    for b in range(NBUF):
        gather(b, b)

    def body(j, carry):
        for b in range(NBUF):
            c = j * NBUF + b
            gather_wait(c, b)
            store(c, b)
        for b in range(NBUF):
            c = j * NBUF + b
            store_wait(c, b)
            gather(c + NBUF, b)
        return carry

    lax.fori_loop(0, MAIN_ITERS, body, 0)

    for b in range(NBUF):
        c = N_CHUNKS - NBUF + b
        gather_wait(c, b)
        store(c, b)
    for b in range(NBUF):
        c = N_CHUNKS - NBUF + b
        store_wait(c, b)


@jax.jit
def _embed(gene_idx, table):
    idx_flat = gene_idx.reshape(-1)
    mesh = plsc.VectorSubcoreMesh(core_axis_name="c", subcore_axis_name="s")
    out = pl.kernel(
        _gather_kernel,
        mesh=mesh,
        out_type=jax.ShapeDtypeStruct((N_SAMPLES, N_GENES_PER, EMBED_DIM),
                                      jnp.float32),
        scratch_types=[
            pltpu.VMEM((PER_W,), jnp.int32),
            pltpu.VMEM((CHUNK, EMBED_DIM), jnp.float32),
            pltpu.VMEM((CHUNK, EMBED_DIM), jnp.float32),
            pltpu.VMEM((CHUNK, EMBED_DIM), jnp.float32),
            pltpu.VMEM((CHUNK, EMBED_DIM), jnp.float32),
            pltpu.SemaphoreType.DMA,
            pltpu.SemaphoreType.DMA,
            pltpu.SemaphoreType.DMA,
            pltpu.SemaphoreType.DMA,
            pltpu.SemaphoreType.DMA,
            pltpu.SemaphoreType.DMA,
            pltpu.SemaphoreType.DMA,
            pltpu.SemaphoreType.DMA,
        ],
        compiler_params=pltpu.CompilerParams(
            use_tc_tiling_on_sc=False, needs_layout_passes=False),
    )(idx_flat, table)
    return out


def kernel(gene_idx, table):
    return _embed(gene_idx, table)
